# Initial kernel scaffold; baseline (speedup 1.0000x reference)
import functools

import jax
import jax.numpy as jnp
from jax import lax
from jax.experimental import pallas as pl
from jax.experimental.pallas import tpu as pltpu
from jax.experimental.pallas import tpu_sc as plsc

N_NODES = 10000
N_EDGES = 320000
D_FEAT = 128
HIDDEN = 3
N_CLASSES = 10

NC = 2
NS = 16
NW = NC * NS
N_PAD = 10240
EPW = N_EDGES // NW
ROWS_PS = N_PAD // NS
PAYW = 4

_MESH = plsc.VectorSubcoreMesh(core_axis_name="c", subcore_axis_name="s")


@jax.jit
def _sc_histogram(dst, ones, zeros1):
    @functools.partial(
        pl.kernel,
        out_type=jax.ShapeDtypeStruct((NC, N_PAD, 1), jnp.float32),
        mesh=_MESH,
        scratch_types=[
            pltpu.VMEM((EPW,), jnp.int32),
            pltpu.VMEM((EPW, 1), jnp.float32),
            pltpu.VMEM_SHARED((N_PAD, 1), jnp.float32),
        ],
    )
    def histo(dst_hbm, ones_hbm, zeros_hbm, out_hbm, idx_v, pay_v, deg_sh):
        c = lax.axis_index("c")
        s = lax.axis_index("s")
        wid = s * NC + c
        pltpu.sync_copy(zeros_hbm.at[pl.ds(s * ROWS_PS, ROWS_PS)],
                        deg_sh.at[pl.ds(s * ROWS_PS, ROWS_PS)])
        pltpu.sync_copy(dst_hbm.at[pl.ds(wid * EPW, EPW)], idx_v)
        pltpu.sync_copy(ones_hbm, pay_v)
        plsc.subcore_barrier()
        pltpu.sync_copy(pay_v, deg_sh.at[idx_v], add=True)
        plsc.subcore_barrier()
        pltpu.sync_copy(deg_sh.at[pl.ds(s * ROWS_PS, ROWS_PS)],
                        out_hbm.at[c].at[pl.ds(s * ROWS_PS, ROWS_PS)])

    return histo(dst, ones, zeros1)


@jax.jit
def _sc_gather_scatter(src, dst, y4, zeros4):
    @functools.partial(
        pl.kernel,
        out_type=jax.ShapeDtypeStruct((NC, N_PAD, PAYW), jnp.float32),
        mesh=_MESH,
        scratch_types=[
            pltpu.VMEM((EPW,), jnp.int32),
            pltpu.VMEM((EPW,), jnp.int32),
            pltpu.VMEM((EPW, PAYW), jnp.float32),
            pltpu.VMEM_SHARED((N_PAD, PAYW), jnp.float32),
        ],
    )
    def gscat(src_hbm, dst_hbm, y_hbm, zeros_hbm, out_hbm,
              si_v, di_v, rows_v, acc_sh):
        c = lax.axis_index("c")
        s = lax.axis_index("s")
        wid = s * NC + c
        pltpu.sync_copy(zeros_hbm.at[pl.ds(s * ROWS_PS, ROWS_PS)],
                        acc_sh.at[pl.ds(s * ROWS_PS, ROWS_PS)])
        pltpu.sync_copy(src_hbm.at[pl.ds(wid * EPW, EPW)], si_v)
        pltpu.sync_copy(dst_hbm.at[pl.ds(wid * EPW, EPW)], di_v)
        pltpu.sync_copy(y_hbm.at[si_v], rows_v)
        plsc.subcore_barrier()
        pltpu.sync_copy(rows_v, acc_sh.at[di_v], add=True)
        plsc.subcore_barrier()
        pltpu.sync_copy(acc_sh.at[pl.ds(s * ROWS_PS, ROWS_PS)],
                        out_hbm.at[c].at[pl.ds(s * ROWS_PS, ROWS_PS)])

    return gscat(src, dst, y4, zeros4)


def _tc_xw(x_pad, W4):
    def body(x_ref, w_ref, xw_ref):
        xw_ref[...] = jnp.dot(x_ref[...], w_ref[...],
                              preferred_element_type=jnp.float32)

    return pl.pallas_call(
        body,
        out_shape=jax.ShapeDtypeStruct((N_PAD, PAYW), jnp.float32),
    )(x_pad, W4)


def _tc_norm(xw4, degp):
    def body(xw_ref, degp_ref, y_ref, dis_ref):
        deg = 1.0 + jnp.sum(degp_ref[...], axis=0)
        dis = lax.rsqrt(deg)
        y_ref[...] = xw_ref[...] * dis
        dis_ref[...] = dis

    return pl.pallas_call(
        body,
        out_shape=[
            jax.ShapeDtypeStruct((N_PAD, PAYW), jnp.float32),
            jax.ShapeDtypeStruct((N_PAD, 1), jnp.float32),
        ],
    )(xw4, degp)


def _tc_final(accp, y4, dis, bg4, wo4, bo):
    def body(accp_ref, y_ref, dis_ref, bg_ref, wo_ref, bo_ref, h_ref, z_ref):
        acc = jnp.sum(accp_ref[...], axis=0)
        agg = dis_ref[...] * (acc + y_ref[...])
        h = jnp.maximum(agg + bg_ref[...], 0.0)
        h_ref[...] = h
        z_ref[...] = jnp.dot(h, wo_ref[...],
                             preferred_element_type=jnp.float32) + bo_ref[...]

    return pl.pallas_call(
        body,
        out_shape=[
            jax.ShapeDtypeStruct((N_PAD, PAYW), jnp.float32),
            jax.ShapeDtypeStruct((N_PAD, N_CLASSES), jnp.float32),
        ],
    )(accp, y4, dis, bg4, wo4, bo)


def kernel(x, edge_index, W_gcn, b_gcn, W_out, b_out):
    src = edge_index[0].astype(jnp.int32)
    dst = edge_index[1].astype(jnp.int32)
    x_pad = jnp.pad(x, ((0, N_PAD - N_NODES), (0, 0)))
    W4 = jnp.pad(W_gcn, ((0, 0), (0, PAYW - HIDDEN)))
    bg4 = jnp.pad(b_gcn, (0, PAYW - HIDDEN)).reshape(1, PAYW)
    wo4 = jnp.pad(W_out, ((0, PAYW - HIDDEN), (0, 0)))
    bo = b_out.reshape(1, N_CLASSES)
    ones = jnp.ones((EPW, 1), jnp.float32)
    zeros1 = jnp.zeros((N_PAD, 1), jnp.float32)
    zeros4 = jnp.zeros((N_PAD, PAYW), jnp.float32)

    degp = _sc_histogram(dst, ones, zeros1)
    xw4 = _tc_xw(x_pad, W4)
    y4, dis = _tc_norm(xw4, degp)
    accp = _sc_gather_scatter(src, dst, y4, zeros4)
    h4, z = _tc_final(accp, y4, dis, bg4, wo4, bo)

    return h4[:N_NODES, :HIDDEN], z[:N_NODES]


# speedup vs baseline: 94.7923x; 94.7923x over previous
import functools

import jax
import jax.numpy as jnp
from jax import lax
from jax.experimental import pallas as pl
from jax.experimental.pallas import tpu as pltpu
from jax.experimental.pallas import tpu_sc as plsc

N_NODES = 10000
N_EDGES = 320000
D_FEAT = 128
HIDDEN = 3
N_CLASSES = 10

NC = 2
NS = 16
NW = NC * NS
VL = 16
N_PAD = 10240
EPW = N_EDGES // NW
NP3 = HIDDEN * N_PAD

_MESH = plsc.VectorSubcoreMesh(core_axis_name="c", subcore_axis_name="s")
_SC_PARAMS = pltpu.CompilerParams(use_tc_tiling_on_sc=False,
                                  needs_layout_passes=False)


@jax.jit
def _sc_histogram(dst):
    @functools.partial(
        pl.kernel,
        out_type=jax.ShapeDtypeStruct((NW * N_PAD,), jnp.float32),
        mesh=_MESH,
        compiler_params=_SC_PARAMS,
        scratch_types=[
            pltpu.VMEM((EPW,), jnp.int32),
            pltpu.VMEM((N_PAD,), jnp.float32),
        ],
    )
    def histo(dst_hbm, out_hbm, idx_v, deg_v):
        c = lax.axis_index("c")
        s = lax.axis_index("s")
        wid = s * NC + c
        pltpu.sync_copy(dst_hbm.at[pl.ds(wid * EPW, EPW)], idx_v)

        @pl.loop(0, N_PAD, step=4 * VL)
        def _(i):
            for u in range(4):
                deg_v[pl.ds(i + u * VL, VL)] = jnp.zeros((VL,), jnp.float32)

        ones = jnp.ones((VL,), jnp.float32)

        @pl.loop(0, EPW, step=VL)
        def _(i):
            d16 = idx_v[pl.ds(i, VL)]
            plsc.addupdate_scatter(deg_v, [d16], ones)

        pltpu.sync_copy(deg_v, out_hbm.at[pl.ds(wid * N_PAD, N_PAD)])

    return histo(dst)


@jax.jit
def _sc_gather_scatter(src, dst, yflat):
    @functools.partial(
        pl.kernel,
        out_type=jax.ShapeDtypeStruct((HIDDEN * NW * N_PAD,), jnp.float32),
        mesh=_MESH,
        compiler_params=_SC_PARAMS,
        scratch_types=[
            pltpu.VMEM((EPW,), jnp.int32),
            pltpu.VMEM((EPW,), jnp.int32),
            pltpu.VMEM((NP3,), jnp.float32),
            pltpu.VMEM((NP3,), jnp.float32),
            pltpu.VMEM_SHARED((NP3,), jnp.float32),
        ],
    )
    def gscat(src_hbm, dst_hbm, y_hbm, out_hbm, si_v, di_v, y_v, acc_v, y_sh):
        c = lax.axis_index("c")
        s = lax.axis_index("s")
        wid = s * NC + c

        @pl.when(s == 0)
        def _():
            pltpu.sync_copy(y_hbm, y_sh)

        pltpu.sync_copy(src_hbm.at[pl.ds(wid * EPW, EPW)], si_v)
        pltpu.sync_copy(dst_hbm.at[pl.ds(wid * EPW, EPW)], di_v)

        @pl.loop(0, NP3, step=4 * VL)
        def _(i):
            for u in range(4):
                acc_v[pl.ds(i + u * VL, VL)] = jnp.zeros((VL,), jnp.float32)

        plsc.subcore_barrier()
        pltpu.sync_copy(y_sh, y_v)

        @pl.loop(0, EPW, step=VL)
        def _(i):
            s16 = si_v[pl.ds(i, VL)]
            d16 = di_v[pl.ds(i, VL)]
            for k in range(HIDDEN):
                v = plsc.load_gather(y_v, [s16 + (k * N_PAD)])
                plsc.addupdate_scatter(acc_v, [d16 + (k * N_PAD)], v)

        for k in range(HIDDEN):
            pltpu.sync_copy(
                acc_v.at[pl.ds(k * N_PAD, N_PAD)],
                out_hbm.at[pl.ds((k * NW + wid) * N_PAD, N_PAD)])

    return gscat(src, dst, yflat)


def _tc_xw(x_pad, W4):
    def body(x_ref, w_ref, xw_ref):
        xw_ref[...] = lax.dot_general(
            w_ref[...], x_ref[...],
            dimension_numbers=(((0,), (1,)), ((), ())),
            preferred_element_type=jnp.float32)

    return pl.pallas_call(
        body,
        out_shape=jax.ShapeDtypeStruct((PAYW_T, N_PAD), jnp.float32),
    )(x_pad, W4)


PAYW_T = 4


def _tc_norm(xwT, degp2):
    def body(xw_ref, degp_ref, y_ref, dis_ref):
        deg = 1.0 + jnp.sum(degp_ref[...], axis=0, keepdims=True)
        dis = lax.rsqrt(deg)
        y_ref[...] = xw_ref[...] * dis
        dis_ref[...] = dis

    return pl.pallas_call(
        body,
        out_shape=[
            jax.ShapeDtypeStruct((PAYW_T, N_PAD), jnp.float32),
            jax.ShapeDtypeStruct((1, N_PAD), jnp.float32),
        ],
    )(xwT, degp2)


def _tc_final(accp, yT, disT, bgT, W_out, boT):
    def body(accp_ref, y_ref, dis_ref, bg_ref, wo_ref, bo_ref, h_ref, z_ref):
        parts = [
            jnp.sum(accp_ref[pl.ds(k * NW, NW), :], axis=0, keepdims=True)
            for k in range(HIDDEN)
        ]
        acc = jnp.concatenate(parts, axis=0)
        agg = dis_ref[...] * (acc + y_ref[pl.ds(0, HIDDEN), :])
        h = jnp.maximum(agg + bg_ref[...], 0.0)
        h_ref[...] = h
        z_ref[...] = lax.dot_general(
            wo_ref[...], h,
            dimension_numbers=(((0,), (0,)), ((), ())),
            preferred_element_type=jnp.float32) + bo_ref[...]

    return pl.pallas_call(
        body,
        out_shape=[
            jax.ShapeDtypeStruct((HIDDEN, N_PAD), jnp.float32),
            jax.ShapeDtypeStruct((N_CLASSES, N_PAD), jnp.float32),
        ],
    )(accp, yT, disT, bgT, W_out, boT)


def kernel(x, edge_index, W_gcn, b_gcn, W_out, b_out):
    src = edge_index[0].astype(jnp.int32)
    dst = edge_index[1].astype(jnp.int32)
    x_pad = jnp.pad(x, ((0, N_PAD - N_NODES), (0, 0)))
    W4 = jnp.pad(W_gcn, ((0, 0), (0, PAYW_T - HIDDEN)))
    bgT = b_gcn.reshape(HIDDEN, 1)
    boT = b_out.reshape(N_CLASSES, 1)

    degp = _sc_histogram(dst)
    xwT = _tc_xw(x_pad, W4)
    degp2 = degp.reshape(NW, N_PAD)
    yT, disT = _tc_norm(xwT, degp2)
    yflat = yT[:HIDDEN].reshape(NP3)
    accp = _sc_gather_scatter(src, dst, yflat)
    accp2 = accp.reshape(HIDDEN * NW, N_PAD)
    hT, zT = _tc_final(accp2, yT, disT, bgT, W_out, boT)

    return hT[:, :N_NODES].T, zT[:, :N_NODES].T
